# Initial kernel scaffold; baseline (speedup 1.0000x reference)
import functools

import jax
import jax.numpy as jnp
from jax import lax
from jax.experimental import pallas as pl
from jax.experimental.pallas import tpu as pltpu
from jax.experimental.pallas import tpu_sc as plsc

N = 10000
E = 320000
D_IN = 128
D_HID = 16
D_OUT = 8

NC = 2
NS = 16
NW = NC * NS
LANE = 128
RPW = 79
EPAD = NW * RPW * LANE
NPAD = 10016
RPT = NPAD // NS
NDEG = 10112
DPT = NDEG // NS

_mesh = plsc.VectorSubcoreMesh(core_axis_name="c", subcore_axis_name="s")


@functools.partial(
    pl.kernel,
    out_type=jax.ShapeDtypeStruct((NC, NDEG), jnp.float32),
    mesh=_mesh,
    scratch_types=[
        pltpu.VMEM((RPW, LANE), jnp.int32),
        pltpu.VMEM((LANE,), jnp.float32),
        pltpu.VMEM_SHARED((NDEG,), jnp.float32),
    ],
)
def _sc_degree(dst_hbm, zeros_hbm, ones_hbm, out_hbm, idx_v, ones_v, acc_s):
    c = lax.axis_index("c")
    s = lax.axis_index("s")
    wid = c * NS + s
    pltpu.sync_copy(ones_hbm, ones_v)
    pltpu.sync_copy(dst_hbm.at[pl.ds(wid * RPW, RPW), :], idx_v)
    pltpu.sync_copy(zeros_hbm.at[pl.ds(s * DPT, DPT)], acc_s.at[pl.ds(s * DPT, DPT)])
    plsc.subcore_barrier()

    @pl.loop(0, RPW)
    def _(j):
        pltpu.sync_copy(ones_v, acc_s.at[idx_v.at[j]], add=True)

    plsc.subcore_barrier()
    pltpu.sync_copy(acc_s.at[pl.ds(s * DPT, DPT)], out_hbm.at[c, pl.ds(s * DPT, DPT)])


def _make_sc_layer(F):
    @functools.partial(
        pl.kernel,
        out_type=jax.ShapeDtypeStruct((NC, NPAD, F), jnp.float32),
        mesh=_mesh,
        scratch_types=[
            pltpu.VMEM((RPW, LANE), jnp.int32),
            pltpu.VMEM((RPW, LANE), jnp.int32),
            pltpu.VMEM((LANE, F), jnp.float32),
            pltpu.VMEM_SHARED((NPAD, F), jnp.float32),
            pltpu.VMEM_SHARED((NPAD, F), jnp.float32),
        ],
    )
    def _sc_layer(tbl_hbm, src_hbm, dst_hbm, zeros_hbm, out_hbm,
                  src_v, dst_v, msg_v, tbl_s, acc_s):
        c = lax.axis_index("c")
        s = lax.axis_index("s")
        wid = c * NS + s
        rows = pl.ds(s * RPT, RPT)
        pltpu.sync_copy(src_hbm.at[pl.ds(wid * RPW, RPW), :], src_v)
        pltpu.sync_copy(dst_hbm.at[pl.ds(wid * RPW, RPW), :], dst_v)
        pltpu.sync_copy(tbl_hbm.at[rows, :], tbl_s.at[rows, :])

        @pl.when(c == 0)
        def _():
            pltpu.sync_copy(tbl_hbm.at[rows, :], acc_s.at[rows, :])

        @pl.when(c == 1)
        def _():
            pltpu.sync_copy(zeros_hbm.at[rows, :], acc_s.at[rows, :])

        plsc.subcore_barrier()

        @pl.loop(0, RPW)
        def _(j):
            pltpu.sync_copy(tbl_s.at[src_v.at[j]], msg_v)
            pltpu.sync_copy(msg_v, acc_s.at[dst_v.at[j]], add=True)

        plsc.subcore_barrier()
        pltpu.sync_copy(acc_s.at[rows, :], out_hbm.at[c, rows, :])

    return _sc_layer


_sc_layer16 = _make_sc_layer(D_HID)
_sc_layer8 = _make_sc_layer(D_OUT)


def _tc_a_body(deg_ref, x_ref, w1_ref, dinv_ref, h1p_ref):
    deg = deg_ref[0] + deg_ref[1] + 1.0
    dinv = lax.rsqrt(deg)
    dinv_ref[...] = dinv
    h = jnp.dot(x_ref[...], w1_ref[...], preferred_element_type=jnp.float32)
    h1p_ref[:N, :] = h * dinv[:N]
    h1p_ref[N:, :] = jnp.zeros((NPAD - N, D_HID), jnp.float32)


def _tc_b_body(p_ref, dinv_ref, b1_ref, w2_ref, h2p_ref):
    sacc = p_ref[0] + p_ref[1]
    dinv = dinv_ref[...]
    z = jnp.maximum(sacc * dinv + b1_ref[...], 0.0)
    h2p_ref[...] = jnp.dot(z, w2_ref[...], preferred_element_type=jnp.float32) * dinv


def _tc_d_body(p_ref, dinv_ref, b2_ref, out_ref):
    sacc = p_ref[0, :N, :] + p_ref[1, :N, :]
    out_ref[...] = sacc * dinv_ref[:N] + b2_ref[...]


def kernel(x, edge_index, W1, b1, W2, b2):
    pad = EPAD - E
    src2d = jnp.concatenate(
        [edge_index[0], jnp.zeros((pad,), jnp.int32)]).reshape(NW * RPW, LANE)
    dst2d = jnp.concatenate(
        [edge_index[1], jnp.full((pad,), N, jnp.int32)]).reshape(NW * RPW, LANE)
    zeros1d = jnp.zeros((NDEG,), jnp.float32)
    ones = jnp.ones((LANE,), jnp.float32)
    zeros16 = jnp.zeros((NPAD, D_HID), jnp.float32)
    zeros8 = jnp.zeros((NPAD, D_OUT), jnp.float32)

    deg_part = _sc_degree(dst2d, zeros1d, ones)
    degp = deg_part[:, :NPAD].reshape(NC, NPAD, 1)

    dinv, h1p = pl.pallas_call(
        _tc_a_body,
        out_shape=(
            jax.ShapeDtypeStruct((NPAD, 1), jnp.float32),
            jax.ShapeDtypeStruct((NPAD, D_HID), jnp.float32),
        ),
    )(degp, x, W1)

    p1 = _sc_layer16(h1p, src2d, dst2d, zeros16)

    h2p = pl.pallas_call(
        _tc_b_body,
        out_shape=jax.ShapeDtypeStruct((NPAD, D_OUT), jnp.float32),
    )(p1, dinv, b1.reshape(1, D_HID), W2)

    p2 = _sc_layer8(h2p, src2d, dst2d, zeros8)

    out = pl.pallas_call(
        _tc_d_body,
        out_shape=jax.ShapeDtypeStruct((N, D_OUT), jnp.float32),
    )(p2, dinv, b2.reshape(1, D_OUT))
    return out


# speedup vs baseline: 54.0800x; 54.0800x over previous
import functools

import jax
import jax.numpy as jnp
from jax import lax
from jax.experimental import pallas as pl
from jax.experimental.pallas import tpu as pltpu
from jax.experimental.pallas import tpu_sc as plsc

N = 10000
E = 320000
D_IN = 128
D_HID = 16
D_OUT = 8

NC = 2
NS = 16
NW = NC * NS
LANE = 128
RPW = 80
EPAD = NW * RPW * LANE
NPAD = 10112
RPT = NPAD // NS
NDEG = NPAD
DPT = NDEG // NS

_mesh = plsc.VectorSubcoreMesh(core_axis_name="c", subcore_axis_name="s")


@functools.partial(
    pl.kernel,
    out_type=jax.ShapeDtypeStruct((NC * NDEG,), jnp.float32),
    mesh=_mesh,
    compiler_params=pltpu.CompilerParams(use_tc_tiling_on_sc=False),
    scratch_types=[
        pltpu.VMEM((RPW, LANE), jnp.int32),
        pltpu.VMEM((LANE,), jnp.float32),
        pltpu.VMEM((DPT,), jnp.float32),
        pltpu.VMEM_SHARED((NDEG,), jnp.float32),
    ],
)
def _sc_degree(dst_hbm, zeros_hbm, ones_hbm, out_hbm, idx_v, ones_v, buf_v, acc_s):
    c = lax.axis_index("c")
    s = lax.axis_index("s")
    wid = c * NS + s
    pltpu.sync_copy(ones_hbm, ones_v)
    pltpu.sync_copy(dst_hbm.at[pl.ds(wid * RPW, RPW), :], idx_v)
    pltpu.sync_copy(zeros_hbm.at[pl.ds(s * DPT, DPT)], buf_v)
    pltpu.sync_copy(buf_v, acc_s.at[pl.ds(s * DPT, DPT)])
    plsc.subcore_barrier()

    @pl.loop(0, RPW)
    def _(j):
        pltpu.sync_copy(ones_v, acc_s.at[idx_v.at[j]], add=True)

    plsc.subcore_barrier()
    pltpu.sync_copy(acc_s.at[pl.ds(s * DPT, DPT)], buf_v)
    pltpu.sync_copy(buf_v, out_hbm.at[pl.ds(c * NDEG + s * DPT, DPT)])


def _make_sc_layer(F):
    @functools.partial(
        pl.kernel,
        out_type=jax.ShapeDtypeStruct((NC, NPAD, F), jnp.float32),
        mesh=_mesh,
        compiler_params=pltpu.CompilerParams(use_tc_tiling_on_sc=False),
        scratch_types=[
            pltpu.VMEM((RPW, LANE), jnp.int32),
            pltpu.VMEM((RPW, LANE), jnp.int32),
            pltpu.VMEM((LANE, F), jnp.float32),
            pltpu.VMEM((RPT, F), jnp.float32),
            pltpu.VMEM_SHARED((NPAD, F), jnp.float32),
            pltpu.VMEM_SHARED((NPAD, F), jnp.float32),
        ],
    )
    def _sc_layer(tbl_hbm, src_hbm, dst_hbm, zeros_hbm, out_hbm,
                  src_v, dst_v, msg_v, buf_v, tbl_s, acc_s):
        c = lax.axis_index("c")
        s = lax.axis_index("s")
        wid = c * NS + s
        rows = pl.ds(s * RPT, RPT)
        pltpu.sync_copy(src_hbm.at[pl.ds(wid * RPW, RPW), :], src_v)
        pltpu.sync_copy(dst_hbm.at[pl.ds(wid * RPW, RPW), :], dst_v)
        pltpu.sync_copy(tbl_hbm.at[rows, :], buf_v)
        pltpu.sync_copy(buf_v, tbl_s.at[rows, :])

        @pl.when(c == 1)
        def _():
            pltpu.sync_copy(zeros_hbm.at[rows, :], buf_v)

        pltpu.sync_copy(buf_v, acc_s.at[rows, :])

        plsc.subcore_barrier()

        @pl.loop(0, RPW)
        def _(j):
            pltpu.sync_copy(tbl_s.at[src_v.at[j]], msg_v)
            pltpu.sync_copy(msg_v, acc_s.at[dst_v.at[j]], add=True)

        plsc.subcore_barrier()
        pltpu.sync_copy(acc_s.at[rows, :], buf_v)
        pltpu.sync_copy(buf_v, out_hbm.at[c, rows, :])

    return _sc_layer


_sc_layer16 = _make_sc_layer(D_HID)
_sc_layer8 = _make_sc_layer(D_OUT)


def _tc_a_body(deg_ref, x_ref, w1_ref, dinv_ref, h1p_ref):
    deg = deg_ref[0] + deg_ref[1] + 1.0
    dinv = lax.rsqrt(deg)
    dinv_ref[...] = dinv
    h = jnp.dot(x_ref[...], w1_ref[...], preferred_element_type=jnp.float32)
    h1p_ref[:N, :] = h * dinv[:N]
    h1p_ref[N:, :] = jnp.zeros((NPAD - N, D_HID), jnp.float32)


def _tc_b_body(p_ref, dinv_ref, b1_ref, w2_ref, h2p_ref):
    sacc = p_ref[0] + p_ref[1]
    dinv = dinv_ref[...]
    z = jnp.maximum(sacc * dinv + b1_ref[...], 0.0)
    h2p_ref[...] = jnp.dot(z, w2_ref[...], preferred_element_type=jnp.float32) * dinv


def _tc_d_body(p_ref, dinv_ref, b2_ref, out_ref):
    sacc = p_ref[0, :N, :] + p_ref[1, :N, :]
    out_ref[...] = sacc * dinv_ref[:N] + b2_ref[...]


def kernel(x, edge_index, W1, b1, W2, b2):
    pad = EPAD - E
    src2d = jnp.concatenate(
        [edge_index[0], jnp.zeros((pad,), jnp.int32)]).reshape(NW * RPW, LANE)
    dst2d = jnp.concatenate(
        [edge_index[1], jnp.full((pad,), N, jnp.int32)]).reshape(NW * RPW, LANE)
    zeros1d = jnp.zeros((NDEG,), jnp.float32)
    ones = jnp.ones((LANE,), jnp.float32)
    zeros16 = jnp.zeros((NPAD, D_HID), jnp.float32)
    zeros8 = jnp.zeros((NPAD, D_OUT), jnp.float32)

    deg_part = _sc_degree(dst2d, zeros1d, ones)
    degp = deg_part.reshape(NC, NPAD, 1)

    dinv, h1p = pl.pallas_call(
        _tc_a_body,
        out_shape=(
            jax.ShapeDtypeStruct((NPAD, 1), jnp.float32),
            jax.ShapeDtypeStruct((NPAD, D_HID), jnp.float32),
        ),
    )(degp, x, W1)

    p1 = _sc_layer16(h1p, src2d, dst2d, zeros16)

    h2p = pl.pallas_call(
        _tc_b_body,
        out_shape=jax.ShapeDtypeStruct((NPAD, D_OUT), jnp.float32),
    )(p1, dinv, b1.reshape(1, D_HID), W2)

    p2 = _sc_layer8(h2p, src2d, dst2d, zeros8)

    out = pl.pallas_call(
        _tc_d_body,
        out_shape=jax.ShapeDtypeStruct((N, D_OUT), jnp.float32),
    )(p2, dinv, b2.reshape(1, D_OUT))
    return out
